# baseline (speedup 1.0000x reference)
import functools

import jax
import jax.numpy as jnp
from jax import lax
from jax.experimental import pallas as pl
from jax.experimental.pallas import tpu as pltpu
from jax.experimental.pallas import tpu_sc as plsc

N_NODES = 10000
N_EDGES = 320000
D = 128

NC = 2
NS = 16
NW = NC * NS
CHUNK = 40
NCHUNK = 250
EDGES_PER_TILE = NCHUNK * CHUNK
E_PAD = NW * EDGES_PER_TILE
NBUF = 5
NGROUP = NCHUNK // NBUF
ZROWS = 32
ACC_ROWS = 10112
ROWS_PER_TILE = ACC_ROWS // NS


def _seg_sum_sc():
    mesh = plsc.VectorSubcoreMesh(core_axis_name="c", subcore_axis_name="s")

    @functools.partial(
        pl.kernel,
        mesh=mesh,
        out_type=jax.ShapeDtypeStruct((2 * ACC_ROWS, D), jnp.float32),
        scratch_types=[
            pltpu.VMEM((NBUF, CHUNK), jnp.int32),
            pltpu.VMEM((NBUF, CHUNK), jnp.int32),
            pltpu.VMEM((NBUF, CHUNK, D), jnp.float32),
            pltpu.VMEM((ZROWS, D), jnp.float32),
            pltpu.VMEM_SHARED((ACC_ROWS, D), jnp.float32),
            pltpu.SemaphoreType.DMA((NBUF,)),
            pltpu.SemaphoreType.DMA((NBUF,)),
            pltpu.SemaphoreType.DMA((NBUF,)),
            pltpu.SemaphoreType.DMA((NBUF,)),
            pltpu.SemaphoreType.DMA,
        ],
    )
    def seg_sum(h_hbm, src_hbm, dst_hbm, out_hbm, sring, dring, rows,
                zbuf, acc, gsem, ssem, xs, xd, zsem):
        cid = lax.axis_index("c")
        sid = lax.axis_index("s")
        wid = sid * NC + cid
        ebase = wid * EDGES_PER_TILE

        def load_sidx(j, b):
            pltpu.async_copy(src_hbm.at[pl.ds(ebase + j * CHUNK, CHUNK)],
                             sring.at[b], xs.at[b])

        def load_didx(j, b):
            pltpu.async_copy(dst_hbm.at[pl.ds(ebase + j * CHUNK, CHUNK)],
                             dring.at[b], xd.at[b])

        def wait_sidx(b):
            pltpu.make_async_copy(src_hbm.at[pl.ds(0, CHUNK)], sring.at[b],
                                  xs.at[b]).wait()

        def wait_didx(b):
            pltpu.make_async_copy(dst_hbm.at[pl.ds(0, CHUNK)], dring.at[b],
                                  xd.at[b]).wait()

        def gather(j, b):
            pltpu.async_copy(h_hbm.at[sring.at[b]], rows.at[b], gsem.at[b])

        def scatter(j, b):
            pltpu.async_copy(rows.at[b], acc.at[dring.at[b]], ssem.at[b],
                             add=True)

        def wait_gather(b):
            pltpu.make_async_copy(h_hbm.at[sring.at[b]], rows.at[b],
                                  gsem.at[b]).wait()

        def wait_scatter(b):
            pltpu.make_async_copy(rows.at[b], acc.at[dring.at[b]],
                                  ssem.at[b]).wait()

        for b in range(NBUF):
            load_sidx(b, b)
            load_didx(b, b)

        def _zrow(r, _):
            for j in range(D // 16):
                zbuf[r, pl.ds(j * 16, 16)] = jnp.zeros((16,), jnp.float32)
            return 0

        lax.fori_loop(0, ZROWS, _zrow, 0)
        rbase = sid * ROWS_PER_TILE
        zcps = [pltpu.async_copy(
                    zbuf, acc.at[pl.ds(rbase + k * ZROWS, ZROWS)], zsem)
                for k in range(ROWS_PER_TILE // ZROWS)]
        zrem = ROWS_PER_TILE % ZROWS
        if zrem:
            zcps.append(pltpu.async_copy(
                zbuf.at[pl.ds(0, zrem)],
                acc.at[pl.ds(rbase + (ROWS_PER_TILE // ZROWS) * ZROWS, zrem)],
                zsem))

        for b in range(NBUF):
            wait_sidx(b)
            gather(b, b)
        for z in zcps:
            z.wait()
        plsc.subcore_barrier()

        def _group(g, _):
            for b in range(NBUF):
                j = g * NBUF + b
                wait_gather(b)

                @pl.when(g < NGROUP - 1)
                def _():
                    load_sidx(j + NBUF, b)
                wait_didx(b)
                scatter(j, b)

            @pl.when(g < NGROUP - 1)
            def _():
                for b in range(NBUF):
                    j = g * NBUF + b
                    wait_scatter(b)
                    load_didx(j + NBUF, b)
                    wait_sidx(b)
                    gather(j + NBUF, b)
            return 0

        lax.fori_loop(0, NGROUP, _group, 0)
        for b in range(NBUF):
            wait_scatter(b)
        plsc.subcore_barrier()

        pltpu.sync_copy(
            acc.at[pl.ds(sid * ROWS_PER_TILE, ROWS_PER_TILE)],
            out_hbm.at[pl.ds(cid * ACC_ROWS + sid * ROWS_PER_TILE,
                             ROWS_PER_TILE)])

    return seg_sum


_SEG_SUM = _seg_sum_sc()


def _mm_body(x_ref, w_ref, o_ref):
    o_ref[...] = jnp.dot(x_ref[...], w_ref[...],
                         preferred_element_type=jnp.float32)


def _matmul(x, w):
    return pl.pallas_call(
        _mm_body,
        out_shape=jax.ShapeDtypeStruct((x.shape[0], w.shape[1]), jnp.float32),
    )(x, w)


def _bn_relu_mm_body(p_ref, b_ref, g_ref, be_ref, w_ref, o_ref):
    s = p_ref[0:N_NODES, :] + p_ref[ACC_ROWS:ACC_ROWS + N_NODES, :] + b_ref[...]
    mean = jnp.mean(s, axis=0, keepdims=True)
    d0 = s - mean
    var = jnp.mean(d0 * d0, axis=0, keepdims=True)
    y = d0 * lax.rsqrt(var + 1e-5) * g_ref[...] + be_ref[...]
    y = jnp.maximum(y, 0.0)
    o_ref[...] = jnp.dot(y, w_ref[...], preferred_element_type=jnp.float32)


def _bn_relu_mm(p, b, g, be, w):
    return pl.pallas_call(
        _bn_relu_mm_body,
        out_shape=jax.ShapeDtypeStruct((N_NODES, D), jnp.float32),
    )(p, b.reshape(1, D), g.reshape(1, D), be.reshape(1, D), w)


def _final_body(p_ref, b_ref, o_ref):
    o_ref[...] = p_ref[0:N_NODES, :] + p_ref[ACC_ROWS:ACC_ROWS + N_NODES, :] + b_ref[...]


def _final(p, b):
    return pl.pallas_call(
        _final_body,
        out_shape=jax.ShapeDtypeStruct((N_NODES, D), jnp.float32),
    )(p, b.reshape(1, D))


def kernel(x, edge_index, W1, b1, W2, b2, W3, b3, g1, be1, g2, be2):
    src = edge_index[0]
    dst = edge_index[1]
    h = _matmul(x, W1)
    p = _SEG_SUM(h, src, dst)
    h = _bn_relu_mm(p, b1, g1, be1, W2)
    p = _SEG_SUM(h, src, dst)
    h = _bn_relu_mm(p, b2, g2, be2, W3)
    p = _SEG_SUM(h, src, dst)
    return _final(p, b3)


# speedup vs baseline: 1.0317x; 1.0317x over previous
import functools

import jax
import jax.numpy as jnp
from jax import lax
from jax.experimental import pallas as pl
from jax.experimental.pallas import tpu as pltpu
from jax.experimental.pallas import tpu_sc as plsc

N_NODES = 10000
N_EDGES = 320000
D = 128

NC = 2
NS = 16
NW = NC * NS
CHUNK = 64
NCHUNK = 156
TAIL = 16
EDGES_PER_TILE = N_EDGES // NW
NBUF = 5
NGROUP = (NCHUNK - 1) // NBUF
ZROWS = 32
ACC_ROWS = 10112
ROWS_PER_TILE = ACC_ROWS // NS


def _seg_sum_sc():
    mesh = plsc.VectorSubcoreMesh(core_axis_name="c", subcore_axis_name="s")

    @functools.partial(
        pl.kernel,
        mesh=mesh,
        out_type=jax.ShapeDtypeStruct((2 * ACC_ROWS, D), jnp.float32),
        scratch_types=[
            pltpu.VMEM((NBUF, CHUNK), jnp.int32),
            pltpu.VMEM((NBUF, CHUNK), jnp.int32),
            pltpu.VMEM((NBUF, CHUNK, D), jnp.float32),
            pltpu.VMEM((ZROWS, D), jnp.float32),
            pltpu.VMEM_SHARED((ACC_ROWS, D), jnp.float32),
            pltpu.SemaphoreType.DMA((NBUF,)),
            pltpu.SemaphoreType.DMA((NBUF,)),
            pltpu.SemaphoreType.DMA((NBUF,)),
            pltpu.SemaphoreType.DMA((NBUF,)),
            pltpu.SemaphoreType.DMA,
            pltpu.VMEM((1, TAIL), jnp.int32),
            pltpu.VMEM((1, TAIL), jnp.int32),
        ],
    )
    def seg_sum(h_hbm, src_hbm, dst_hbm, out_hbm, sring, dring, rows,
                zbuf, acc, gsem, ssem, xs, xd, zsem, tsr, tdr):
        cid = lax.axis_index("c")
        sid = lax.axis_index("s")
        wid = sid * NC + cid
        ebase = wid * EDGES_PER_TILE

        def load_sidx(j, b):
            pltpu.async_copy(src_hbm.at[pl.ds(ebase + j * CHUNK, CHUNK)],
                             sring.at[b], xs.at[b])

        def load_didx(j, b):
            pltpu.async_copy(dst_hbm.at[pl.ds(ebase + j * CHUNK, CHUNK)],
                             dring.at[b], xd.at[b])

        def wait_sidx(b):
            pltpu.make_async_copy(src_hbm.at[pl.ds(0, CHUNK)], sring.at[b],
                                  xs.at[b]).wait()

        def wait_didx(b):
            pltpu.make_async_copy(dst_hbm.at[pl.ds(0, CHUNK)], dring.at[b],
                                  xd.at[b]).wait()

        def gather(j, b):
            pltpu.async_copy(h_hbm.at[sring.at[b]], rows.at[b], gsem.at[b])

        def scatter(j, b):
            pltpu.async_copy(rows.at[b], acc.at[dring.at[b]], ssem.at[b],
                             add=True)

        def wait_gather(b):
            pltpu.make_async_copy(h_hbm.at[sring.at[b]], rows.at[b],
                                  gsem.at[b]).wait()

        def wait_scatter(b):
            pltpu.make_async_copy(rows.at[b], acc.at[dring.at[b]],
                                  ssem.at[b]).wait()

        for b in range(NBUF):
            load_sidx(b, b)
            load_didx(b, b)

        def _zrow(r, _):
            for j in range(D // 16):
                zbuf[r, pl.ds(j * 16, 16)] = jnp.zeros((16,), jnp.float32)
            return 0

        lax.fori_loop(0, ZROWS, _zrow, 0)
        rbase = sid * ROWS_PER_TILE
        zcps = [pltpu.async_copy(
                    zbuf, acc.at[pl.ds(rbase + k * ZROWS, ZROWS)], zsem)
                for k in range(ROWS_PER_TILE // ZROWS)]
        zrem = ROWS_PER_TILE % ZROWS
        if zrem:
            zcps.append(pltpu.async_copy(
                zbuf.at[pl.ds(0, zrem)],
                acc.at[pl.ds(rbase + (ROWS_PER_TILE // ZROWS) * ZROWS, zrem)],
                zsem))

        for b in range(NBUF):
            wait_sidx(b)
            gather(b, b)
        for z in zcps:
            z.wait()
        plsc.subcore_barrier()

        def _group(g, _):
            for b in range(NBUF):
                j = g * NBUF + b
                wait_gather(b)

                @pl.when(g < NGROUP - 1)
                def _():
                    load_sidx(j + NBUF, b)
                wait_didx(b)
                scatter(j, b)

            @pl.when(g < NGROUP - 1)
            def _():
                for b in range(NBUF):
                    j = g * NBUF + b
                    wait_scatter(b)
                    load_didx(j + NBUF, b)
                    wait_sidx(b)
                    gather(j + NBUF, b)
            return 0

        lax.fori_loop(0, NGROUP, _group, 0)
        for b in range(NBUF):
            wait_scatter(b)

        last = NCHUNK - 1
        toff = ebase + NCHUNK * CHUNK
        pltpu.async_copy(src_hbm.at[pl.ds(ebase + last * CHUNK, CHUNK)],
                         sring.at[0], xs.at[0])
        pltpu.async_copy(dst_hbm.at[pl.ds(ebase + last * CHUNK, CHUNK)],
                         dring.at[0], xd.at[0])
        pltpu.async_copy(src_hbm.at[pl.ds(toff, TAIL)], tsr.at[0], xs.at[1])
        pltpu.async_copy(dst_hbm.at[pl.ds(toff, TAIL)], tdr.at[0], xd.at[1])
        wait_sidx(0)
        gather(last, 0)
        pltpu.make_async_copy(src_hbm.at[pl.ds(0, TAIL)], tsr.at[0],
                              xs.at[1]).wait()
        pltpu.async_copy(h_hbm.at[tsr.at[0]], rows.at[1, pl.ds(0, TAIL)],
                         gsem.at[1])
        wait_gather(0)
        wait_didx(0)
        scatter(last, 0)
        pltpu.make_async_copy(h_hbm.at[tsr.at[0]], rows.at[1, pl.ds(0, TAIL)],
                              gsem.at[1]).wait()
        pltpu.make_async_copy(dst_hbm.at[pl.ds(0, TAIL)], tdr.at[0],
                              xd.at[1]).wait()
        pltpu.async_copy(rows.at[1, pl.ds(0, TAIL)], acc.at[tdr.at[0]],
                         ssem.at[1], add=True)
        wait_scatter(0)
        pltpu.make_async_copy(rows.at[1, pl.ds(0, TAIL)], acc.at[tdr.at[0]],
                              ssem.at[1]).wait()
        plsc.subcore_barrier()

        pltpu.sync_copy(
            acc.at[pl.ds(sid * ROWS_PER_TILE, ROWS_PER_TILE)],
            out_hbm.at[pl.ds(cid * ACC_ROWS + sid * ROWS_PER_TILE,
                             ROWS_PER_TILE)])

    return seg_sum


_SEG_SUM = _seg_sum_sc()


def _mm_body(x_ref, w_ref, o_ref):
    o_ref[...] = jnp.dot(x_ref[...], w_ref[...],
                         preferred_element_type=jnp.float32)


def _matmul(x, w):
    return pl.pallas_call(
        _mm_body,
        out_shape=jax.ShapeDtypeStruct((x.shape[0], w.shape[1]), jnp.float32),
    )(x, w)


def _bn_relu_mm_body(p_ref, b_ref, g_ref, be_ref, w_ref, o_ref):
    s = p_ref[0:N_NODES, :] + p_ref[ACC_ROWS:ACC_ROWS + N_NODES, :] + b_ref[...]
    mean = jnp.mean(s, axis=0, keepdims=True)
    d0 = s - mean
    var = jnp.mean(d0 * d0, axis=0, keepdims=True)
    y = d0 * lax.rsqrt(var + 1e-5) * g_ref[...] + be_ref[...]
    y = jnp.maximum(y, 0.0)
    o_ref[...] = jnp.dot(y, w_ref[...], preferred_element_type=jnp.float32)


def _bn_relu_mm(p, b, g, be, w):
    return pl.pallas_call(
        _bn_relu_mm_body,
        out_shape=jax.ShapeDtypeStruct((N_NODES, D), jnp.float32),
    )(p, b.reshape(1, D), g.reshape(1, D), be.reshape(1, D), w)


def _final_body(p_ref, b_ref, o_ref):
    o_ref[...] = p_ref[0:N_NODES, :] + p_ref[ACC_ROWS:ACC_ROWS + N_NODES, :] + b_ref[...]


def _final(p, b):
    return pl.pallas_call(
        _final_body,
        out_shape=jax.ShapeDtypeStruct((N_NODES, D), jnp.float32),
    )(p, b.reshape(1, D))


def kernel(x, edge_index, W1, b1, W2, b2, W3, b3, g1, be1, g2, be2):
    src = edge_index[0]
    dst = edge_index[1]
    h = _matmul(x, W1)
    p = _SEG_SUM(h, src, dst)
    h = _bn_relu_mm(p, b1, g1, be1, W2)
    p = _SEG_SUM(h, src, dst)
    h = _bn_relu_mm(p, b2, g2, be2, W3)
    p = _SEG_SUM(h, src, dst)
    return _final(p, b3)
